# baseline (speedup 1.0000x reference)
import functools

import jax
import jax.numpy as jnp
from jax import lax
from jax.experimental import pallas as pl
from jax.experimental.pallas import tpu as pltpu
from jax.experimental.pallas import tpu_sc as plsc

_NE = 8192
_D = 256
_NT = 8192
_COMMIT = 0.25

_TB = 512
_CB = 1024


def _dist_body(x_ref, w_ref, dist_ref, mv_ref, mi_ref, mv_s, mi_s):
    j = pl.program_id(1)
    nj = pl.num_programs(1)
    x = x_ref[...]
    w = w_ref[...]
    x2 = jnp.sum(x * x, axis=1, keepdims=True)
    w2 = jnp.sum(w * w, axis=1)
    mm = lax.dot_general(x, w, (((1,), (1,)), ((), ())),
                         preferred_element_type=jnp.float32)
    dist = (x2 + w2[None, :]) - 2.0 * mm
    dist_ref[...] = dist

    tmin = jnp.min(dist, axis=1)
    cols = lax.broadcasted_iota(jnp.int32, dist.shape, 1)
    targ = jnp.min(jnp.where(dist == tmin[:, None], cols, _NE),
                   axis=1) + j * _CB

    @pl.when(j == 0)
    def _():
        mv_s[...] = tmin
        mi_s[...] = targ

    @pl.when(j > 0)
    def _():
        better = tmin < mv_s[...]
        mv_s[...] = jnp.where(better, tmin, mv_s[...])
        mi_s[...] = jnp.where(better, targ, mi_s[...])

    @pl.when(j == nj - 1)
    def _():
        mv_ref[...] = mv_s[...]
        mi_ref[...] = mi_s[...]


def _distances_argmin(flat_x, weight):
    return pl.pallas_call(
        _dist_body,
        grid=(_NT // _TB, _NE // _CB),
        in_specs=[
            pl.BlockSpec((_TB, _D), lambda i, j: (i, 0)),
            pl.BlockSpec((_CB, _D), lambda i, j: (j, 0)),
        ],
        out_specs=[
            pl.BlockSpec((_TB, _CB), lambda i, j: (i, j)),
            pl.BlockSpec((_TB,), lambda i, j: (i,)),
            pl.BlockSpec((_TB,), lambda i, j: (i,)),
        ],
        out_shape=[
            jax.ShapeDtypeStruct((_NT, _NE), jnp.float32),
            jax.ShapeDtypeStruct((_NT,), jnp.float32),
            jax.ShapeDtypeStruct((_NT,), jnp.int32),
        ],
        scratch_shapes=[
            pltpu.VMEM((_TB,), jnp.float32),
            pltpu.VMEM((_TB,), jnp.int32),
        ],
    )(flat_x, weight)


def _enc_body(mi_ref, mv_ref, enc_ref, loss_ref, ppl_ref, cnt_s, acc_s):
    j = pl.program_id(0)
    i = pl.program_id(1)
    nj = pl.num_programs(0)
    ni = pl.num_programs(1)
    idx = mi_ref[...]
    cols = j * _CB + lax.broadcasted_iota(jnp.int32, (_TB, _CB), 1)
    onehot = (idx[:, None] == cols).astype(jnp.float32)
    enc_ref[...] = onehot
    colsum = jnp.sum(onehot, axis=0)

    @pl.when(i == 0)
    def _():
        cnt_s[...] = colsum

    @pl.when(i > 0)
    def _():
        cnt_s[...] = cnt_s[...] + colsum

    @pl.when(jnp.logical_and(j == 0, i == 0))
    def _():
        acc_s[0] = 0.0
        acc_s[1] = 0.0

    @pl.when(j == 0)
    def _():
        acc_s[0] = acc_s[0] + jnp.sum(mv_ref[...])

    @pl.when(i == ni - 1)
    def _():
        p = cnt_s[...] * (1.0 / _NT)
        acc_s[1] = acc_s[1] + jnp.sum(p * jnp.log(p + 1e-10))

    @pl.when(jnp.logical_and(j == nj - 1, i == ni - 1))
    def _():
        loss_ref[0, 0] = (1.0 + _COMMIT) * acc_s[0] * (1.0 / (_NT * _D))
        ppl_ref[0, 0] = jnp.exp(-acc_s[1])


def _encodings_stats(minidx, minval):
    return pl.pallas_call(
        _enc_body,
        grid=(_NE // _CB, _NT // _TB),
        in_specs=[
            pl.BlockSpec((_TB,), lambda j, i: (i,)),
            pl.BlockSpec((_TB,), lambda j, i: (i,)),
        ],
        out_specs=[
            pl.BlockSpec((_TB, _CB), lambda j, i: (i, j)),
            pl.BlockSpec(memory_space=pltpu.SMEM),
            pl.BlockSpec(memory_space=pltpu.SMEM),
        ],
        out_shape=[
            jax.ShapeDtypeStruct((_NT, _NE), jnp.float32),
            jax.ShapeDtypeStruct((1, 1), jnp.float32),
            jax.ShapeDtypeStruct((1, 1), jnp.float32),
        ],
        scratch_shapes=[
            pltpu.VMEM((_CB,), jnp.float32),
            pltpu.SMEM((2,), jnp.float32),
        ],
    )(minidx, minval)


def _sc_gather(weight, minidx):
    info = plsc.get_sparse_core_info()
    nw = info.num_cores * info.num_subcores
    bpw = _NT // nw
    nchunks = bpw // 128
    idx2 = minidx.reshape(_NT // 128, 128)
    mesh = plsc.VectorSubcoreMesh(core_axis_name="c", subcore_axis_name="s")

    @functools.partial(
        pl.kernel, mesh=mesh,
        out_type=jax.ShapeDtypeStruct((_NT, _D), jnp.float32),
        scratch_types=[
            pltpu.VMEM((nchunks, 128), jnp.int32),
            pltpu.VMEM((bpw, _D), jnp.float32),
            pltpu.SemaphoreType.DMA,
        ],
    )
    def k(w_hbm, idx_hbm, out_hbm, idx_v, rows_v, sem):
        wid = lax.axis_index("s") * info.num_cores + lax.axis_index("c")
        pltpu.sync_copy(idx_hbm.at[pl.ds(wid * nchunks, nchunks)], idx_v)
        for c in range(nchunks):
            pltpu.async_copy(w_hbm.at[idx_v.at[c]],
                             rows_v.at[pl.ds(c * 128, 128)], sem).wait()
        pltpu.sync_copy(rows_v, out_hbm.at[pl.ds(wid * bpw, bpw)])

    return k(weight, idx2)


def kernel(inputs, weight):
    flat_x = jnp.transpose(inputs, (1, 2, 0)).reshape(_NT, _D)
    dist, minval, minidx = _distances_argmin(flat_x, weight)
    quant = _sc_gather(weight, minidx)
    enc, loss, ppl = _encodings_stats(minidx, minval)
    out_q = jnp.transpose(quant.reshape(16, 512, _D), (2, 0, 1))
    return (loss.reshape(()),
            out_q,
            ppl.reshape(()),
            enc.reshape(256, 512, 512),
            dist.reshape(256, 512, 512),
            minidx[:, None])


# speedup vs baseline: 1.1074x; 1.1074x over previous
import functools

import jax
import jax.numpy as jnp
from jax import lax
from jax.experimental import pallas as pl
from jax.experimental.pallas import tpu as pltpu
from jax.experimental.pallas import tpu_sc as plsc

_NE = 8192
_D = 256
_NT = 8192
_COMMIT = 0.25

_TB = 512
_CB = 2048
_EB = 4096


def _dist_body(x_ref, w_ref, dist_ref, mv_ref, mi_ref, mv_s, mi_s):
    j = pl.program_id(1)
    nj = pl.num_programs(1)
    x = x_ref[...]
    w = w_ref[pl.ds(j * _CB, _CB), :]
    x2 = jnp.sum(x * x, axis=1, keepdims=True)
    w2 = jnp.sum(w * w, axis=1)
    mm = lax.dot_general(x, w, (((1,), (1,)), ((), ())),
                         preferred_element_type=jnp.float32)
    dist = (x2 + w2[None, :]) - 2.0 * mm
    dist_ref[...] = dist

    tmin = jnp.min(dist, axis=1)
    cols = lax.broadcasted_iota(jnp.int32, dist.shape, 1)
    targ = jnp.min(jnp.where(dist == tmin[:, None], cols, _NE),
                   axis=1) + j * _CB

    @pl.when(j == 0)
    def _():
        mv_s[...] = tmin
        mi_s[...] = targ

    @pl.when(j > 0)
    def _():
        better = tmin < mv_s[...]
        mv_s[...] = jnp.where(better, tmin, mv_s[...])
        mi_s[...] = jnp.where(better, targ, mi_s[...])

    @pl.when(j == nj - 1)
    def _():
        mv_ref[...] = mv_s[...]
        mi_ref[...] = mi_s[...]


def _distances_argmin(flat_x, weight):
    return pl.pallas_call(
        _dist_body,
        grid=(_NT // _TB, _NE // _CB),
        in_specs=[
            pl.BlockSpec((_TB, _D), lambda i, j: (i, 0)),
            pl.BlockSpec((_NE, _D), lambda i, j: (0, 0)),
        ],
        out_specs=[
            pl.BlockSpec((_TB, _CB), lambda i, j: (i, j)),
            pl.BlockSpec((_TB,), lambda i, j: (i,)),
            pl.BlockSpec((_TB,), lambda i, j: (i,)),
        ],
        out_shape=[
            jax.ShapeDtypeStruct((_NT, _NE), jnp.float32),
            jax.ShapeDtypeStruct((_NT,), jnp.float32),
            jax.ShapeDtypeStruct((_NT,), jnp.int32),
        ],
        scratch_shapes=[
            pltpu.VMEM((_TB,), jnp.float32),
            pltpu.VMEM((_TB,), jnp.int32),
        ],
    )(flat_x, weight)


def _enc_body(mi_ref, mv_ref, enc_ref, loss_ref, ppl_ref, cnt_s, acc_s):
    j = pl.program_id(0)
    i = pl.program_id(1)
    nj = pl.num_programs(0)
    ni = pl.num_programs(1)
    idx = mi_ref[...]
    cols = j * _EB + lax.broadcasted_iota(jnp.int32, (_TB, _EB), 1)
    onehot = (idx[:, None] == cols).astype(jnp.float32)
    enc_ref[...] = onehot
    colsum = jnp.sum(onehot, axis=0)

    @pl.when(i == 0)
    def _():
        cnt_s[...] = colsum

    @pl.when(i > 0)
    def _():
        cnt_s[...] = cnt_s[...] + colsum

    @pl.when(jnp.logical_and(j == 0, i == 0))
    def _():
        acc_s[0] = 0.0
        acc_s[1] = 0.0

    @pl.when(j == 0)
    def _():
        acc_s[0] = acc_s[0] + jnp.sum(mv_ref[...])

    @pl.when(i == ni - 1)
    def _():
        p = cnt_s[...] * (1.0 / _NT)
        acc_s[1] = acc_s[1] + jnp.sum(p * jnp.log(p + 1e-10))

    @pl.when(jnp.logical_and(j == nj - 1, i == ni - 1))
    def _():
        loss_ref[0, 0] = (1.0 + _COMMIT) * acc_s[0] * (1.0 / (_NT * _D))
        ppl_ref[0, 0] = jnp.exp(-acc_s[1])


def _encodings_stats(minidx, minval):
    return pl.pallas_call(
        _enc_body,
        grid=(_NE // _EB, _NT // _TB),
        in_specs=[
            pl.BlockSpec((_TB,), lambda j, i: (i,)),
            pl.BlockSpec((_TB,), lambda j, i: (i,)),
        ],
        out_specs=[
            pl.BlockSpec((_TB, _EB), lambda j, i: (i, j)),
            pl.BlockSpec(memory_space=pltpu.SMEM),
            pl.BlockSpec(memory_space=pltpu.SMEM),
        ],
        out_shape=[
            jax.ShapeDtypeStruct((_NT, _NE), jnp.float32),
            jax.ShapeDtypeStruct((1, 1), jnp.float32),
            jax.ShapeDtypeStruct((1, 1), jnp.float32),
        ],
        scratch_shapes=[
            pltpu.VMEM((_EB,), jnp.float32),
            pltpu.SMEM((2,), jnp.float32),
        ],
    )(minidx, minval)


def _sc_gather(weight, minidx):
    info = plsc.get_sparse_core_info()
    nw = info.num_cores * info.num_subcores
    bpw = _NT // nw
    nchunks = bpw // 128
    idx2 = minidx.reshape(_NT // 128, 128)
    mesh = plsc.VectorSubcoreMesh(core_axis_name="c", subcore_axis_name="s")

    @functools.partial(
        pl.kernel, mesh=mesh,
        out_type=jax.ShapeDtypeStruct((_NT, _D), jnp.float32),
        scratch_types=[
            pltpu.VMEM((nchunks, 128), jnp.int32),
            pltpu.VMEM((bpw, _D), jnp.float32),
            pltpu.SemaphoreType.DMA,
        ],
    )
    def k(w_hbm, idx_hbm, out_hbm, idx_v, rows_v, sem):
        wid = lax.axis_index("s") * info.num_cores + lax.axis_index("c")
        pltpu.sync_copy(idx_hbm.at[pl.ds(wid * nchunks, nchunks)], idx_v)
        for c in range(nchunks):
            pltpu.async_copy(w_hbm.at[idx_v.at[c]],
                             rows_v.at[pl.ds(c * 128, 128)], sem).wait()
        pltpu.sync_copy(rows_v, out_hbm.at[pl.ds(wid * bpw, bpw)])

    return k(weight, idx2)


def kernel(inputs, weight):
    flat_x = jnp.transpose(inputs, (1, 2, 0)).reshape(_NT, _D)
    dist, minval, minidx = _distances_argmin(flat_x, weight)
    quant = _sc_gather(weight, minidx)
    enc, loss, ppl = _encodings_stats(minidx, minval)
    out_q = jnp.transpose(quant.reshape(16, 512, _D), (2, 0, 1))
    return (loss.reshape(()),
            out_q,
            ppl.reshape(()),
            enc.reshape(256, 512, 512),
            dist.reshape(256, 512, 512),
            minidx[:, None])
